# baseline (speedup 1.0000x reference)
import jax
import jax.numpy as jnp
from jax.experimental import pallas as pl

N_CTR = 1024
M_PTS = 2048
BN = 8


def _fluid_solid_kernel(mask_ref, cdT_ref, cT_ref,
                        fW0T_ref, fW1T_ref, fW2T_ref, fW3T_ref,
                        fb0_ref, fb1_ref, fb2_ref, fb3_ref,
                        sW0T_ref, sW1T_ref, sW2T_ref, sW3T_ref,
                        sb0_ref, sb1_ref, sb2_ref, sb3_ref,
                        out_ref):
    hi = jax.lax.Precision.HIGHEST
    cdT = cdT_ref[...]
    cT = cT_ref[0]
    mask_blk = mask_ref[...]

    ptype = cdT[6:7, :]
    tf_row = 1.0 - ptype
    ts_row = ptype

    fW0T = fW0T_ref[...]
    sW0T = sW0T_ref[...]

    afT = jnp.dot(fW0T[:, 0:6], cdT[0:6, :], precision=hi)
    asT = jnp.dot(sW0T[:, 0:3], cdT[0:3, :], precision=hi)

    posc = cT[0:3, :]
    velc = cT[3:6, :]
    bfT = (jnp.dot(fW0T[:, 0:3], -posc, precision=hi)
           + jnp.dot(fW0T[:, 6:9], velc, precision=hi) + fb0_ref[...])
    bsT = (jnp.dot(sW0T[:, 0:3], -posc, precision=hi)
           + jnp.dot(sW0T[:, 3:6], velc, precision=hi) + sb0_ref[...])

    fW1T = fW1T_ref[...]
    fW2T = fW2T_ref[...]
    fW3T = fW3T_ref[...]
    sW1T = sW1T_ref[...]
    sW2T = sW2T_ref[...]
    sW3T = sW3T_ref[...]
    fb1 = fb1_ref[...]
    fb2 = fb2_ref[...]
    fb3 = fb3_ref[...]
    sb1 = sb1_ref[...]
    sb2 = sb2_ref[...]
    sb3 = sb3_ref[...]

    for i in range(BN):
        hf = jnp.tanh(afT + bfT[:, i:i + 1])
        hf = jnp.tanh(jnp.dot(fW1T, hf, precision=hi) + fb1)
        hf = jnp.tanh(jnp.dot(fW2T, hf, precision=hi) + fb2)

        hs = jnp.tanh(asT + bsT[:, i:i + 1])
        hs = jnp.tanh(jnp.dot(sW1T, hs, precision=hi) + sb1)
        hs = jnp.tanh(jnp.dot(sW2T, hs, precision=hi) + sb2)

        wf = mask_blk[i:i + 1, :] * tf_row
        ws = mask_blk[i:i + 1, :] * ts_row

        sf = jnp.sum(hf * wf, axis=1, keepdims=True)
        ss = jnp.sum(hs * ws, axis=1, keepdims=True)
        cf = jnp.sum(wf, axis=1, keepdims=True)
        cs = jnp.sum(ws, axis=1, keepdims=True)

        out_ref[0, :, i:i + 1] = (jnp.dot(fW3T, sf, precision=hi) + fb3 * cf
                                  + jnp.dot(sW3T, ss, precision=hi) + sb3 * cs)


def kernel(mask, center_particle, current_data,
           fW0, fb0, fW1, fb1, fW2, fb2, fW3, fb3,
           sW0, sb0, sW1, sb1, sW2, sb2, sW3, sb3):
    n, m = mask.shape
    maskf = mask.astype(jnp.float32)
    cdT = current_data.T
    grid = n // BN
    cTb = center_particle.T.reshape(6, grid, BN).transpose(1, 0, 2)

    col = lambda b: b.reshape(-1, 1)

    full = lambda shape: pl.BlockSpec(shape, lambda i: (0, 0))
    outT = pl.pallas_call(
        _fluid_solid_kernel,
        grid=(grid,),
        in_specs=[
            pl.BlockSpec((BN, m), lambda i: (i, 0)),
            full((7, m)),
            pl.BlockSpec((1, 6, BN), lambda i: (i, 0, 0)),
            full((18, 9)), full((9, 18)), full((6, 9)), full((3, 6)),
            full((18, 1)), full((9, 1)), full((6, 1)), full((3, 1)),
            full((18, 6)), full((9, 18)), full((6, 9)), full((3, 6)),
            full((18, 1)), full((9, 1)), full((6, 1)), full((3, 1)),
        ],
        out_specs=pl.BlockSpec((1, 3, BN), lambda i: (i, 0, 0)),
        out_shape=jax.ShapeDtypeStruct((grid, 3, BN), jnp.float32),
    )(maskf, cdT, cTb,
      fW0.T, fW1.T, fW2.T, fW3.T, col(fb0), col(fb1), col(fb2), col(fb3),
      sW0.T, sW1.T, sW2.T, sW3.T, col(sb0), col(sb1), col(sb2), col(sb3))
    return outT.transpose(0, 2, 1).reshape(n, 3)


# speedup vs baseline: 1.7533x; 1.7533x over previous
import jax
import jax.numpy as jnp
from jax.experimental import pallas as pl

N_CTR = 1024
M_PTS = 2048
BN = 8


def _fluid_solid_kernel(mask_ref, cdT_ref, cT_ref,
                        fW0T_ref, fW1T_ref, fW2T_ref, fW3T_ref,
                        fb0_ref, fb1_ref, fb2_ref, fb3_ref,
                        sW0T_ref, sW1T_ref, sW2T_ref, sW3T_ref,
                        sb0_ref, sb1_ref, sb2_ref, sb3_ref,
                        out_ref):
    hi = jax.lax.Precision.DEFAULT
    cdT = cdT_ref[...]
    cT = cT_ref[0]
    mask_blk = mask_ref[...]

    ptype = cdT[6:7, :]
    tf_row = 1.0 - ptype
    ts_row = ptype

    fW0T = fW0T_ref[...]
    sW0T = sW0T_ref[...]

    afT = jnp.dot(fW0T[:, 0:6], cdT[0:6, :], precision=hi)
    asT = jnp.dot(sW0T[:, 0:3], cdT[0:3, :], precision=hi)

    posc = cT[0:3, :]
    velc = cT[3:6, :]
    bfT = (jnp.dot(fW0T[:, 0:3], -posc, precision=hi)
           + jnp.dot(fW0T[:, 6:9], velc, precision=hi) + fb0_ref[...])
    bsT = (jnp.dot(sW0T[:, 0:3], -posc, precision=hi)
           + jnp.dot(sW0T[:, 3:6], velc, precision=hi) + sb0_ref[...])

    fW1T = fW1T_ref[...]
    fW2T = fW2T_ref[...]
    fW3T = fW3T_ref[...]
    sW1T = sW1T_ref[...]
    sW2T = sW2T_ref[...]
    sW3T = sW3T_ref[...]
    fb1 = fb1_ref[...]
    fb2 = fb2_ref[...]
    fb3 = fb3_ref[...]
    sb1 = sb1_ref[...]
    sb2 = sb2_ref[...]
    sb3 = sb3_ref[...]

    for i in range(BN):
        hf = jnp.tanh(afT + bfT[:, i:i + 1])
        hf = jnp.tanh(jnp.dot(fW1T, hf, precision=hi) + fb1)
        hf = jnp.tanh(jnp.dot(fW2T, hf, precision=hi) + fb2)

        hs = jnp.tanh(asT + bsT[:, i:i + 1])
        hs = jnp.tanh(jnp.dot(sW1T, hs, precision=hi) + sb1)
        hs = jnp.tanh(jnp.dot(sW2T, hs, precision=hi) + sb2)

        wf = mask_blk[i:i + 1, :] * tf_row
        ws = mask_blk[i:i + 1, :] * ts_row

        sf = jnp.sum(hf * wf, axis=1, keepdims=True)
        ss = jnp.sum(hs * ws, axis=1, keepdims=True)
        cf = jnp.sum(wf, axis=1, keepdims=True)
        cs = jnp.sum(ws, axis=1, keepdims=True)

        out_ref[0, :, i:i + 1] = (jnp.dot(fW3T, sf, precision=hi) + fb3 * cf
                                  + jnp.dot(sW3T, ss, precision=hi) + sb3 * cs)


def kernel(mask, center_particle, current_data,
           fW0, fb0, fW1, fb1, fW2, fb2, fW3, fb3,
           sW0, sb0, sW1, sb1, sW2, sb2, sW3, sb3):
    n, m = mask.shape
    maskf = mask.astype(jnp.float32)
    cdT = current_data.T
    grid = n // BN
    cTb = center_particle.T.reshape(6, grid, BN).transpose(1, 0, 2)

    col = lambda b: b.reshape(-1, 1)

    full = lambda shape: pl.BlockSpec(shape, lambda i: (0, 0))
    outT = pl.pallas_call(
        _fluid_solid_kernel,
        grid=(grid,),
        in_specs=[
            pl.BlockSpec((BN, m), lambda i: (i, 0)),
            full((7, m)),
            pl.BlockSpec((1, 6, BN), lambda i: (i, 0, 0)),
            full((18, 9)), full((9, 18)), full((6, 9)), full((3, 6)),
            full((18, 1)), full((9, 1)), full((6, 1)), full((3, 1)),
            full((18, 6)), full((9, 18)), full((6, 9)), full((3, 6)),
            full((18, 1)), full((9, 1)), full((6, 1)), full((3, 1)),
        ],
        out_specs=pl.BlockSpec((1, 3, BN), lambda i: (i, 0, 0)),
        out_shape=jax.ShapeDtypeStruct((grid, 3, BN), jnp.float32),
    )(maskf, cdT, cTb,
      fW0.T, fW1.T, fW2.T, fW3.T, col(fb0), col(fb1), col(fb2), col(fb3),
      sW0.T, sW1.T, sW2.T, sW3.T, col(sb0), col(sb1), col(sb2), col(sb3))
    return outT.transpose(0, 2, 1).reshape(n, 3)


# speedup vs baseline: 4.5380x; 2.5882x over previous
import jax
import jax.numpy as jnp
from jax.experimental import pallas as pl
from jax.scipy.linalg import block_diag

BN = 8
GRP = 4


def _fluid_solid_kernel(mask_ref, cdT_ref, cT_ref,
                        Wa_ref, Wc_ref, b0_ref,
                        W1g_ref, b1_ref, W2g_ref, b2_ref, W3g_ref,
                        fb3_ref, sb3_ref, out_ref):
    cdT = cdT_ref[...]
    m = cdT.shape[1]
    ptype = cdT[6:7, :]
    tf_row = 1.0 - ptype
    ts_row = ptype

    afs = jnp.dot(Wa_ref[...], cdT[0:6, :])
    bfs = jnp.dot(Wc_ref[...], cT_ref[0]) + b0_ref[...]

    mask_blk = mask_ref[...]
    wf = mask_blk * tf_row
    ws = mask_blk * ts_row

    outs = []
    for g in range(BN // GRP):
        x0 = jnp.concatenate(
            [afs + bfs[:, g * GRP + i:g * GRP + i + 1] for i in range(GRP)],
            axis=0)
        x0 = jnp.tanh(x0)
        x1 = jnp.tanh(jnp.dot(W1g_ref[...], x0) + b1_ref[...])
        x2 = jnp.tanh(jnp.dot(W2g_ref[...], x1) + b2_ref[...])

        wfg = wf[g * GRP:(g + 1) * GRP]
        wsg = ws[g * GRP:(g + 1) * GRP]
        wsel = jnp.concatenate(
            [jnp.broadcast_to(wfg[:, None, :], (GRP, 6, m)),
             jnp.broadcast_to(wsg[:, None, :], (GRP, 6, m))],
            axis=1).reshape(12 * GRP, m)
        s = jnp.sum(x2 * wsel, axis=1, keepdims=True)
        outs.append(jnp.dot(W3g_ref[...], s).reshape(GRP, 3))
    out = jnp.concatenate(outs, axis=0)

    cf = jnp.sum(wf, axis=1, keepdims=True)
    cs = jnp.sum(ws, axis=1, keepdims=True)
    out_ref[0] = out + cf * fb3_ref[...] + cs * sb3_ref[...]


def kernel(mask, center_particle, current_data,
           fW0, fb0, fW1, fb1, fW2, fb2, fW3, fb3,
           sW0, sb0, sW1, sb1, sW2, sb2, sW3, sb3):
    n, m = mask.shape
    maskf = mask.astype(jnp.float32)
    cdT = current_data.T
    grid = n // BN
    cTb = center_particle.T.reshape(6, grid, BN).transpose(1, 0, 2)

    z3 = jnp.zeros((18, 3), jnp.float32)
    Wa = jnp.concatenate([fW0[0:6].T,
                          jnp.concatenate([sW0[0:3].T, z3], axis=1)], axis=0)
    Wc = jnp.concatenate(
        [jnp.concatenate([-fW0[0:3].T, fW0[6:9].T], axis=1),
         jnp.concatenate([-sW0[0:3].T, sW0[3:6].T], axis=1)], axis=0)
    b0 = jnp.concatenate([fb0, sb0]).reshape(36, 1)
    W1fs = block_diag(fW1.T, sW1.T)
    W1g = block_diag(*([W1fs] * GRP))
    b1 = jnp.tile(jnp.concatenate([fb1, sb1]), GRP).reshape(18 * GRP, 1)
    W2fs = block_diag(fW2.T, sW2.T)
    W2g = block_diag(*([W2fs] * GRP))
    b2 = jnp.tile(jnp.concatenate([fb2, sb2]), GRP).reshape(12 * GRP, 1)
    W3fs = jnp.concatenate([fW3.T, sW3.T], axis=1)
    W3g = block_diag(*([W3fs] * GRP))

    full = lambda shape: pl.BlockSpec(shape, lambda i: tuple(0 for _ in shape))
    out = pl.pallas_call(
        _fluid_solid_kernel,
        grid=(grid,),
        in_specs=[
            pl.BlockSpec((BN, m), lambda i: (i, 0)),
            full((7, m)),
            pl.BlockSpec((1, 6, BN), lambda i: (i, 0, 0)),
            full((36, 6)), full((36, 6)), full((36, 1)),
            full((18 * GRP, 36 * GRP)), full((18 * GRP, 1)),
            full((12 * GRP, 18 * GRP)), full((12 * GRP, 1)),
            full((12, 12 * GRP)),
            full((1, 3)), full((1, 3)),
        ],
        out_specs=pl.BlockSpec((1, BN, 3), lambda i: (i, 0, 0)),
        out_shape=jax.ShapeDtypeStruct((grid, BN, 3), jnp.float32),
    )(maskf, cdT, cTb, Wa, Wc, b0, W1g, b1, W2g, b2, W3g,
      fb3.reshape(1, 3), sb3.reshape(1, 3))
    return out.reshape(n, 3)
